# Initial kernel scaffold; baseline (speedup 1.0000x reference)
import jax
import jax.numpy as jnp
from jax.experimental import pallas as pl


def kernel(x, edge_index, W1, b1, W2, b2):
    raise NotImplementedError("write your pallas kernel here")


# speedup vs baseline: 13.0075x; 13.0075x over previous
import functools

import jax
import jax.numpy as jnp
from jax import lax
from jax.experimental import pallas as pl
from jax.experimental.pallas import tpu as pltpu
from jax.experimental.pallas import tpu_sc as plsc

N = 10000
D = 128
E = 320000
NC = 2
NS = 16
NW = NC * NS
CHUNK = 128
EPW_CHUNKS = 79
EPW = EPW_CHUNKS * CHUNK
E_PAD = NW * EPW
N_PAD = 10240
RPS = N_PAD // NS
PAD_SRC = N
PAD_DST = N + 128
BLK = 256
GRID = N_PAD // BLK

_mesh = plsc.VectorSubcoreMesh(core_axis_name="c", subcore_axis_name="s")


def _deg_body(dst_hbm, deg_out, dst_v, ones_v, zvec_v, deg_sh):
    cid = lax.axis_index("c")
    sid = lax.axis_index("s")
    wid = cid * NS + sid
    for c in range(CHUNK // 16):
        ones_v[pl.ds(c * 16, 16)] = jnp.ones((16,), jnp.float32)
    for c in range(RPS // 16):
        zvec_v[pl.ds(c * 16, 16)] = jnp.zeros((16,), jnp.float32)
    pltpu.sync_copy(zvec_v, deg_sh.at[pl.ds(sid * RPS, RPS)])
    plsc.subcore_barrier()
    pltpu.sync_copy(dst_hbm.at[wid], dst_v)

    def body(j, carry):
        pltpu.sync_copy(ones_v, deg_sh.at[dst_v.at[j]], add=True)
        return carry

    lax.fori_loop(0, EPW_CHUNKS, body, 0)
    plsc.subcore_barrier()
    pltpu.sync_copy(deg_sh.at[pl.ds(sid * RPS, RPS)],
                    deg_out.at[cid, pl.ds(sid * RPS, RPS)])


_deg_call = pl.kernel(
    _deg_body,
    out_type=jax.ShapeDtypeStruct((NC, N_PAD), jnp.float32),
    mesh=_mesh,
    scratch_types=[
        pltpu.VMEM((EPW_CHUNKS, CHUNK), jnp.int32),
        pltpu.VMEM((CHUNK,), jnp.float32),
        pltpu.VMEM((RPS,), jnp.float32),
        pltpu.VMEM_SHARED((N_PAD,), jnp.float32),
    ],
)


def _agg_body(g_hbm, src_hbm, dst_hbm, out_hbm, src_v, dst_v, rowbuf, agg_sh):
    cid = lax.axis_index("c")
    sid = lax.axis_index("s")
    wid = cid * NS + sid

    def zb(i, carry):
        for c in range(D // 16):
            rowbuf[i, pl.ds(c * 16, 16)] = jnp.zeros((16,), jnp.float32)
        return carry

    lax.fori_loop(0, CHUNK, zb, 0)
    for k in range(RPS // CHUNK):
        pltpu.sync_copy(rowbuf, agg_sh.at[pl.ds(sid * RPS + k * CHUNK, CHUNK)])
    plsc.subcore_barrier()

    pltpu.sync_copy(src_hbm.at[wid], src_v)
    pltpu.sync_copy(dst_hbm.at[wid], dst_v)

    def body(j, carry):
        pltpu.sync_copy(g_hbm.at[src_v.at[j]], rowbuf)
        pltpu.sync_copy(rowbuf, agg_sh.at[dst_v.at[j]], add=True)
        return carry

    lax.fori_loop(0, EPW_CHUNKS, body, 0)
    plsc.subcore_barrier()
    for k in range(RPS // CHUNK):
        pltpu.sync_copy(agg_sh.at[pl.ds(sid * RPS + k * CHUNK, CHUNK)],
                        out_hbm.at[cid, pl.ds(sid * RPS + k * CHUNK, CHUNK)])


_agg_call = pl.kernel(
    _agg_body,
    out_type=jax.ShapeDtypeStruct((NC, N_PAD, D), jnp.float32),
    mesh=_mesh,
    scratch_types=[
        pltpu.VMEM((EPW_CHUNKS, CHUNK), jnp.int32),
        pltpu.VMEM((EPW_CHUNKS, CHUNK), jnp.int32),
        pltpu.VMEM((CHUNK, D), jnp.float32),
        pltpu.VMEM_SHARED((N_PAD, D), jnp.float32),
    ],
)


def _scale(degt, valid):
    d = (degt[:, 0:1] + degt[:, 1:2] + 1.0) * valid
    return jnp.where(d > 0, lax.rsqrt(d), 0.0)


def _k1_body(x_ref, w_ref, degt_ref, valid_ref, o_ref):
    s = _scale(degt_ref[...], valid_ref[...])
    o_ref[...] = jnp.dot(x_ref[...], w_ref[...],
                         preferred_element_type=jnp.float32) * s


def _k2_body(a_ref, b_ref, g_ref, degt_ref, valid_ref, bias_ref, w_ref, o_ref):
    s = _scale(degt_ref[...], valid_ref[...])
    pre = (a_ref[...] + b_ref[...] + g_ref[...]) * s + bias_ref[...]
    z = jnp.maximum(pre, 0.0)
    o_ref[...] = jnp.dot(z, w_ref[...], preferred_element_type=jnp.float32) * s


def _k3_body(a_ref, b_ref, g_ref, degt_ref, valid_ref, bias_ref, o_ref):
    s = _scale(degt_ref[...], valid_ref[...])
    o_ref[...] = (a_ref[...] + b_ref[...] + g_ref[...]) * s + bias_ref[...]


_row_spec = pl.BlockSpec((BLK, D), lambda i: (i, 0))
_degt_spec = pl.BlockSpec((BLK, 2), lambda i: (i, 0))
_valid_spec = pl.BlockSpec((BLK, 1), lambda i: (i, 0))
_w_spec = pl.BlockSpec((D, D), lambda i: (0, 0))
_bias_spec = pl.BlockSpec((1, D), lambda i: (0, 0))
_out_shape = jax.ShapeDtypeStruct((N_PAD, D), jnp.float32)

_k1_call = pl.pallas_call(
    _k1_body, grid=(GRID,),
    in_specs=[_row_spec, _w_spec, _degt_spec, _valid_spec],
    out_specs=_row_spec, out_shape=_out_shape)

_k2_call = pl.pallas_call(
    _k2_body, grid=(GRID,),
    in_specs=[_row_spec, _row_spec, _row_spec, _degt_spec, _valid_spec,
              _bias_spec, _w_spec],
    out_specs=_row_spec, out_shape=_out_shape)

_k3_call = pl.pallas_call(
    _k3_body, grid=(GRID,),
    in_specs=[_row_spec, _row_spec, _row_spec, _degt_spec, _valid_spec,
              _bias_spec],
    out_specs=_row_spec, out_shape=_out_shape)


def kernel(x, edge_index, W1, b1, W2, b2):
    src = edge_index[0].astype(jnp.int32)
    dst = edge_index[1].astype(jnp.int32)
    pad_e = E_PAD - E
    srcp = jnp.concatenate(
        [src, jnp.full((pad_e,), PAD_SRC, jnp.int32)]).reshape(
            NW, EPW_CHUNKS, CHUNK)
    dstp = jnp.concatenate(
        [dst, jnp.full((pad_e,), PAD_DST, jnp.int32)]).reshape(
            NW, EPW_CHUNKS, CHUNK)
    xp = jnp.pad(x, ((0, N_PAD - N), (0, 0)))
    valid = (jnp.arange(N_PAD) < N).astype(jnp.float32)[:, None]

    degp = _deg_call(dstp)
    degt = degp.T
    g1 = _k1_call(xp, W1, degt, valid)
    agg1 = _agg_call(g1, srcp, dstp)
    g2 = _k2_call(agg1[0], agg1[1], g1, degt, valid, b1.reshape(1, D), W2)
    agg2 = _agg_call(g2, srcp, dstp)
    outp = _k3_call(agg2[0], agg2[1], g2, degt, valid, b2.reshape(1, D))
    return outp[:N]
